# Initial kernel scaffold; baseline (speedup 1.0000x reference)
import jax
import jax.numpy as jnp
from jax.experimental import pallas as pl


def kernel(x, edge_index, Wl1, Wr1, b1, Wl2, Wr2, b2, Wout, bout):
    raise NotImplementedError("write your pallas kernel here")


# speedup vs baseline: 2.6949x; 2.6949x over previous
import functools

import jax
import jax.numpy as jnp
from jax import lax
from jax.experimental import pallas as pl
from jax.experimental.pallas import tpu as pltpu
from jax.experimental.pallas import tpu_sc as plsc

N, E, D, P = 10000, 320000, 128, 12

NC, NS, LANES = 2, 16, 16
NW = NC * NS
CHUNK = 128
CPT = 80
E_PAD = NW * CPT * CHUNK
ROWS_PER_SUB = 640
N_PAD = NS * ROWS_PER_SUB
DUMMY_ROW = N_PAD - 8
CW = 128

_mesh = plsc.VectorSubcoreMesh(core_axis_name="c", subcore_axis_name="s")


def _make_sc_aggregate():
    out_types = jax.ShapeDtypeStruct((NC, N_PAD, D), jnp.float32)
    scratch = [pltpu.VMEM((CHUNK,), jnp.int32),
               pltpu.VMEM((CHUNK,), jnp.int32),
               pltpu.VMEM((CHUNK, D), jnp.float32),
               pltpu.VMEM_SHARED((N_PAD, D), jnp.float32),
               pltpu.SemaphoreType.DMA]

    @functools.partial(pl.kernel, out_type=out_types, mesh=_mesh,
                       scratch_types=scratch)
    def sc_kernel(h_hbm, src_hbm, dst_hbm, pp, idx_s, idx_d, rows,
                  acc, sem):
        cid = lax.axis_index("c")
        sid = lax.axis_index("s")
        wid = sid * NC + cid
        base = sid * ROWS_PER_SUB

        zvec = jnp.zeros((LANES,), jnp.float32)

        @pl.loop(0, CHUNK)
        def _(r):
            for c in range(0, D, LANES):
                rows[r, pl.ds(c, LANES)] = zvec

        for k in range(ROWS_PER_SUB // CHUNK):
            pltpu.sync_copy(rows, acc.at[pl.ds(base + k * CHUNK, CHUNK)])
        plsc.subcore_barrier()

        ebase = wid * CPT * CHUNK

        @pl.loop(0, CPT)
        def _(i):
            off = ebase + i * CHUNK
            pltpu.sync_copy(src_hbm.at[pl.ds(off, CHUNK)], idx_s)
            pltpu.sync_copy(dst_hbm.at[pl.ds(off, CHUNK)], idx_d)
            pltpu.async_copy(h_hbm.at[idx_s], rows, sem).wait()
            pltpu.sync_copy(rows, acc.at[idx_d], add=True)

        plsc.subcore_barrier()

        for k in range(ROWS_PER_SUB // CHUNK):
            sl = pl.ds(base + k * CHUNK, CHUNK)
            pltpu.sync_copy(acc.at[sl], pp.at[cid, sl])

    return sc_kernel


def _make_sc_count():
    out_types = jax.ShapeDtypeStruct((NC, N_PAD, CW), jnp.float32)
    scratch = [pltpu.VMEM((CHUNK,), jnp.int32),
               pltpu.VMEM((CHUNK, CW), jnp.float32),
               pltpu.VMEM((CHUNK, CW), jnp.float32),
               pltpu.VMEM_SHARED((N_PAD, CW), jnp.float32)]

    @functools.partial(pl.kernel, out_type=out_types, mesh=_mesh,
                       scratch_types=scratch)
    def sc_kernel(dst_hbm, cc, idx_d, ones, zc, cnt):
        cid = lax.axis_index("c")
        sid = lax.axis_index("s")
        wid = sid * NC + cid
        base = sid * ROWS_PER_SUB

        @pl.loop(0, CHUNK)
        def _(r):
            for c in range(0, CW, LANES):
                ones[r, pl.ds(c, LANES)] = jnp.ones((LANES,), jnp.float32)
                zc[r, pl.ds(c, LANES)] = jnp.zeros((LANES,), jnp.float32)

        for k in range(ROWS_PER_SUB // CHUNK):
            pltpu.sync_copy(zc, cnt.at[pl.ds(base + k * CHUNK, CHUNK)])
        plsc.subcore_barrier()

        ebase = wid * CPT * CHUNK

        @pl.loop(0, CPT)
        def _(i):
            pltpu.sync_copy(dst_hbm.at[pl.ds(ebase + i * CHUNK, CHUNK)], idx_d)
            pltpu.sync_copy(ones, cnt.at[idx_d], add=True)

        plsc.subcore_barrier()

        for k in range(ROWS_PER_SUB // CHUNK):
            sl = pl.ds(base + k * CHUNK, CHUNK)
            pltpu.sync_copy(cnt.at[sl], cc.at[cid, sl])

    return sc_kernel


_sc_aggregate = _make_sc_aggregate()
_sc_count = _make_sc_count()

BM = 1000


def _tc_layer_body(p0, p1, c0, c1, h, wl, wr, b, o):
    invc = 1.0 / jnp.maximum(c0[:, 0:1] + c1[:, 0:1], 1.0)
    mean = (p0[...] + p1[...]) * invc
    hp = lax.dot_general(mean, wl[...], (((1,), (1,)), ((), ())),
                         precision=lax.Precision.HIGHEST)
    hp += lax.dot_general(h[...], wr[...], (((1,), (1,)), ((), ())),
                          precision=lax.Precision.HIGHEST)
    o[...] = jnp.maximum(hp + b[...], 0.0)


def _tc_layer2_body(p0, p1, c0, c1, h, wl, wr, b, wout, bout, o):
    invc = 1.0 / jnp.maximum(c0[:, 0:1] + c1[:, 0:1], 1.0)
    mean = (p0[...] + p1[...]) * invc
    hp = lax.dot_general(mean, wl[...], (((1,), (1,)), ((), ())),
                         precision=lax.Precision.HIGHEST)
    hp += lax.dot_general(h[...], wr[...], (((1,), (1,)), ((), ())),
                          precision=lax.Precision.HIGHEST)
    h2 = jnp.maximum(hp + b[...], 0.0)
    o[...] = lax.dot_general(h2, wout[...], (((1,), (1,)), ((), ())),
                             precision=lax.Precision.HIGHEST) + bout[...]


def _row_spec(bm, d):
    return pl.BlockSpec((bm, d), lambda i: (i, 0))


def _full_spec(shape):
    return pl.BlockSpec(shape, lambda i: tuple(0 for _ in shape))


def _tc_layer1(p0, p1, c0, c1, h, wl, wr, b):
    grid = (N // BM,)
    return pl.pallas_call(
        _tc_layer_body,
        grid=grid,
        in_specs=[_row_spec(BM, D), _row_spec(BM, D),
                  _row_spec(BM, CW), _row_spec(BM, CW),
                  _row_spec(BM, D),
                  _full_spec((D, D)), _full_spec((D, D)),
                  _full_spec((1, D))],
        out_specs=_row_spec(BM, D),
        out_shape=jax.ShapeDtypeStruct((N, D), jnp.float32),
    )(p0, p1, c0, c1, h, wl, wr, b.reshape(1, D))


def _tc_layer2(p0, p1, c0, c1, h, wl, wr, b, wout, bout):
    grid = (N // BM,)
    return pl.pallas_call(
        _tc_layer2_body,
        grid=grid,
        in_specs=[_row_spec(BM, D), _row_spec(BM, D),
                  _row_spec(BM, CW), _row_spec(BM, CW),
                  _row_spec(BM, D),
                  _full_spec((D, D)), _full_spec((D, D)),
                  _full_spec((1, D)),
                  _full_spec((P, D)), _full_spec((1, P))],
        out_specs=_row_spec(BM, P),
        out_shape=jax.ShapeDtypeStruct((N, P), jnp.float32),
    )(p0, p1, c0, c1, h, wl, wr, b.reshape(1, D), wout, bout.reshape(1, P))


def kernel(x, edge_index, Wl1, Wr1, b1, Wl2, Wr2, b2, Wout, bout):
    src = edge_index[0]
    dst = edge_index[1]
    pad = E_PAD - E
    src_p = jnp.concatenate([src, jnp.zeros((pad,), jnp.int32)])
    dst_p = jnp.concatenate([dst, jnp.full((pad,), DUMMY_ROW, jnp.int32)])

    cc = _sc_count(dst_p)
    pp = _sc_aggregate(x, src_p, dst_p)
    h1 = _tc_layer1(pp[0], pp[1], cc[0], cc[1], x, Wl1, Wr1, b1)
    qq = _sc_aggregate(h1, src_p, dst_p)
    out = _tc_layer2(qq[0], qq[1], cc[0], cc[1], h1, Wl2, Wr2, b2, Wout, bout)
    return out


# speedup vs baseline: 3.2179x; 1.1941x over previous
import functools

import jax
import jax.numpy as jnp
from jax import lax
from jax.experimental import pallas as pl
from jax.experimental.pallas import tpu as pltpu
from jax.experimental.pallas import tpu_sc as plsc

N, E, D, P = 10000, 320000, 128, 12

NC, NS, LANES = 2, 16, 16
NW = NC * NS
CHUNK = 128
CPT = 80
E_PAD = NW * CPT * CHUNK
ROWS_PER_SUB = 640
N_PAD = NS * ROWS_PER_SUB
DUMMY_ROW = N_PAD - 8
CW = 128

_mesh = plsc.VectorSubcoreMesh(core_axis_name="c", subcore_axis_name="s")


def _make_sc_aggregate():
    out_types = jax.ShapeDtypeStruct((NC, N_PAD, D), jnp.float32)
    scratch = [pltpu.VMEM((CPT * CHUNK,), jnp.int32),
               pltpu.VMEM((CHUNK,), jnp.int32),
               pltpu.VMEM((CHUNK,), jnp.int32),
               pltpu.VMEM((CHUNK, D), jnp.float32),
               pltpu.VMEM((CHUNK, D), jnp.float32),
               pltpu.VMEM_SHARED((N_PAD, D), jnp.float32),
               pltpu.SemaphoreType.DMA,
               pltpu.SemaphoreType.DMA,
               pltpu.SemaphoreType.DMA,
               pltpu.SemaphoreType.DMA]

    @functools.partial(pl.kernel, out_type=out_types, mesh=_mesh,
                       scratch_types=scratch)
    def sc_kernel(h_hbm, src_hbm, dst_hbm, z_hbm, pp, srcall, dst0, dst1,
                  rows0, rows1, acc, sg0, sg1, sd0, sd1):
        cid = lax.axis_index("c")
        sid = lax.axis_index("s")
        wid = sid * NC + cid
        base = sid * ROWS_PER_SUB
        ebase = wid * CPT * CHUNK

        dstb = (dst0, dst1)
        rows = (rows0, rows1)
        sg = (sg0, sg1)
        sd = (sd0, sd1)

        pltpu.sync_copy(src_hbm.at[pl.ds(ebase, CPT * CHUNK)], srcall)
        pltpu.sync_copy(z_hbm, acc.at[pl.ds(base, ROWS_PER_SUB)])

        def issue(i, b):
            pltpu.async_copy(dst_hbm.at[pl.ds(ebase + i * CHUNK, CHUNK)],
                             dstb[b], sd[b])
            pltpu.async_copy(h_hbm.at[srcall.at[pl.ds(i * CHUNK, CHUNK)]],
                             rows[b], sg[b])

        issue(0, 0)
        issue(1, 1)
        plsc.subcore_barrier()

        @pl.loop(0, CPT // 2)
        def _(j):
            i0 = j * 2
            for b in range(2):
                i = i0 + b
                pltpu.make_async_copy(dst_hbm.at[pl.ds(0, CHUNK)],
                                      dstb[b], sd[b]).wait()
                pltpu.make_async_copy(h_hbm.at[pl.ds(0, CHUNK)],
                                      rows[b], sg[b]).wait()
                pltpu.sync_copy(rows[b], acc.at[dstb[b]], add=True)

                @pl.when(i + 2 < CPT)
                def _():
                    issue(i + 2, b)

        plsc.subcore_barrier()
        pltpu.sync_copy(acc.at[pl.ds(base, ROWS_PER_SUB)],
                        pp.at[cid, pl.ds(base, ROWS_PER_SUB)])

    return sc_kernel


def _make_sc_count():
    out_types = jax.ShapeDtypeStruct((NC, N_PAD, CW), jnp.float32)
    scratch = [pltpu.VMEM((CHUNK,), jnp.int32),
               pltpu.VMEM((CHUNK, CW), jnp.float32),
               pltpu.VMEM((CHUNK, CW), jnp.float32),
               pltpu.VMEM_SHARED((N_PAD, CW), jnp.float32)]

    @functools.partial(pl.kernel, out_type=out_types, mesh=_mesh,
                       scratch_types=scratch)
    def sc_kernel(dst_hbm, cc, idx_d, ones, zc, cnt):
        cid = lax.axis_index("c")
        sid = lax.axis_index("s")
        wid = sid * NC + cid
        base = sid * ROWS_PER_SUB

        @pl.loop(0, CHUNK)
        def _(r):
            for c in range(0, CW, LANES):
                ones[r, pl.ds(c, LANES)] = jnp.ones((LANES,), jnp.float32)
                zc[r, pl.ds(c, LANES)] = jnp.zeros((LANES,), jnp.float32)

        for k in range(ROWS_PER_SUB // CHUNK):
            pltpu.sync_copy(zc, cnt.at[pl.ds(base + k * CHUNK, CHUNK)])
        plsc.subcore_barrier()

        ebase = wid * CPT * CHUNK

        @pl.loop(0, CPT)
        def _(i):
            pltpu.sync_copy(dst_hbm.at[pl.ds(ebase + i * CHUNK, CHUNK)], idx_d)
            pltpu.sync_copy(ones, cnt.at[idx_d], add=True)

        plsc.subcore_barrier()

        for k in range(ROWS_PER_SUB // CHUNK):
            sl = pl.ds(base + k * CHUNK, CHUNK)
            pltpu.sync_copy(cnt.at[sl], cc.at[cid, sl])

    return sc_kernel


_sc_aggregate = _make_sc_aggregate()
_sc_count = _make_sc_count()

BM = 1000


def _tc_layer_body(p0, p1, c0, c1, h, wl, wr, b, o):
    invc = 1.0 / jnp.maximum(c0[:, 0:1] + c1[:, 0:1], 1.0)
    mean = (p0[...] + p1[...]) * invc
    hp = lax.dot_general(mean, wl[...], (((1,), (1,)), ((), ())),
                         precision=lax.Precision.HIGHEST)
    hp += lax.dot_general(h[...], wr[...], (((1,), (1,)), ((), ())),
                          precision=lax.Precision.HIGHEST)
    o[...] = jnp.maximum(hp + b[...], 0.0)


def _tc_layer2_body(p0, p1, c0, c1, h, wl, wr, b, wout, bout, o):
    invc = 1.0 / jnp.maximum(c0[:, 0:1] + c1[:, 0:1], 1.0)
    mean = (p0[...] + p1[...]) * invc
    hp = lax.dot_general(mean, wl[...], (((1,), (1,)), ((), ())),
                         precision=lax.Precision.HIGHEST)
    hp += lax.dot_general(h[...], wr[...], (((1,), (1,)), ((), ())),
                          precision=lax.Precision.HIGHEST)
    h2 = jnp.maximum(hp + b[...], 0.0)
    o[...] = lax.dot_general(h2, wout[...], (((1,), (1,)), ((), ())),
                             precision=lax.Precision.HIGHEST) + bout[...]


def _row_spec(bm, d):
    return pl.BlockSpec((bm, d), lambda i: (i, 0))


def _full_spec(shape):
    return pl.BlockSpec(shape, lambda i: tuple(0 for _ in shape))


def _tc_layer1(p0, p1, c0, c1, h, wl, wr, b):
    grid = (N // BM,)
    return pl.pallas_call(
        _tc_layer_body,
        grid=grid,
        in_specs=[_row_spec(BM, D), _row_spec(BM, D),
                  _row_spec(BM, CW), _row_spec(BM, CW),
                  _row_spec(BM, D),
                  _full_spec((D, D)), _full_spec((D, D)),
                  _full_spec((1, D))],
        out_specs=_row_spec(BM, D),
        out_shape=jax.ShapeDtypeStruct((N, D), jnp.float32),
    )(p0, p1, c0, c1, h, wl, wr, b.reshape(1, D))


def _tc_layer2(p0, p1, c0, c1, h, wl, wr, b, wout, bout):
    grid = (N // BM,)
    return pl.pallas_call(
        _tc_layer2_body,
        grid=grid,
        in_specs=[_row_spec(BM, D), _row_spec(BM, D),
                  _row_spec(BM, CW), _row_spec(BM, CW),
                  _row_spec(BM, D),
                  _full_spec((D, D)), _full_spec((D, D)),
                  _full_spec((1, D)),
                  _full_spec((P, D)), _full_spec((1, P))],
        out_specs=_row_spec(BM, P),
        out_shape=jax.ShapeDtypeStruct((N, P), jnp.float32),
    )(p0, p1, c0, c1, h, wl, wr, b.reshape(1, D), wout, bout.reshape(1, P))


def kernel(x, edge_index, Wl1, Wr1, b1, Wl2, Wr2, b2, Wout, bout):
    src = edge_index[0]
    dst = edge_index[1]
    pad = E_PAD - E
    src_p = jnp.concatenate([src, jnp.zeros((pad,), jnp.int32)])
    dst_p = jnp.concatenate([dst, jnp.full((pad,), DUMMY_ROW, jnp.int32)])

    zrows = jnp.zeros((ROWS_PER_SUB, D), jnp.float32)
    cc = _sc_count(dst_p)
    pp = _sc_aggregate(x, src_p, dst_p, zrows)
    h1 = _tc_layer1(pp[0], pp[1], cc[0], cc[1], x, Wl1, Wr1, b1)
    qq = _sc_aggregate(h1, src_p, dst_p, zrows)
    out = _tc_layer2(qq[0], qq[1], cc[0], cc[1], h1, Wl2, Wr2, b2, Wout, bout)
    return out


# speedup vs baseline: 8.7840x; 2.7297x over previous
import functools

import jax
import jax.numpy as jnp
from jax import lax
from jax.experimental import pallas as pl
from jax.experimental.pallas import tpu as pltpu
from jax.experimental.pallas import tpu_sc as plsc

N, E, D, P = 10000, 320000, 128, 12

NC, NS, LANES = 2, 16, 16
NW = NC * NS
CHUNK = 128
CPT = 80
E_PAD = NW * CPT * CHUNK
ROWS_PER_SUB = 640
N_PAD = NS * ROWS_PER_SUB
DUMMY_ROW = N_PAD - 8
CW = 128

_mesh = plsc.VectorSubcoreMesh(core_axis_name="c", subcore_axis_name="s")


def _make_sc_aggregate():
    out_types = jax.ShapeDtypeStruct((NC, N_PAD, D), jnp.float32)
    scratch = [pltpu.VMEM((CPT * CHUNK,), jnp.int32),
               pltpu.VMEM((CHUNK,), jnp.int32),
               pltpu.VMEM((CHUNK,), jnp.int32),
               pltpu.VMEM((CHUNK, D), jnp.float32),
               pltpu.VMEM((CHUNK, D), jnp.float32),
               pltpu.VMEM_SHARED((N_PAD, D), jnp.float32),
               pltpu.SemaphoreType.DMA,
               pltpu.SemaphoreType.DMA,
               pltpu.SemaphoreType.DMA,
               pltpu.SemaphoreType.DMA]

    @functools.partial(pl.kernel, out_type=out_types, mesh=_mesh,
                       scratch_types=scratch)
    def sc_kernel(h_hbm, src_hbm, dst_hbm, z_hbm, pp, srcall, dst0, dst1,
                  rows0, rows1, acc, sg0, sg1, sd0, sd1):
        cid = lax.axis_index("c")
        sid = lax.axis_index("s")
        wid = sid * NC + cid
        base = sid * ROWS_PER_SUB
        ebase = wid * CPT * CHUNK

        dstb = (dst0, dst1)
        rows = (rows0, rows1)
        sg = (sg0, sg1)
        sd = (sd0, sd1)

        pltpu.sync_copy(src_hbm.at[pl.ds(ebase, CPT * CHUNK)], srcall)
        pltpu.sync_copy(z_hbm, acc.at[pl.ds(base, ROWS_PER_SUB)])

        def issue(i, b):
            pltpu.async_copy(dst_hbm.at[pl.ds(ebase + i * CHUNK, CHUNK)],
                             dstb[b], sd[b])
            pltpu.async_copy(h_hbm.at[srcall.at[pl.ds(i * CHUNK, CHUNK)]],
                             rows[b], sg[b])

        issue(0, 0)
        issue(1, 1)
        plsc.subcore_barrier()

        @pl.loop(0, CPT // 2)
        def _(j):
            i0 = j * 2
            for b in range(2):
                i = i0 + b
                pltpu.make_async_copy(dst_hbm.at[pl.ds(0, CHUNK)],
                                      dstb[b], sd[b]).wait()
                pltpu.make_async_copy(h_hbm.at[pl.ds(0, CHUNK)],
                                      rows[b], sg[b]).wait()
                pltpu.sync_copy(rows[b], acc.at[dstb[b]], add=True)

                @pl.when(i + 2 < CPT)
                def _():
                    issue(i + 2, b)

        plsc.subcore_barrier()
        pltpu.sync_copy(acc.at[pl.ds(base, ROWS_PER_SUB)],
                        pp.at[cid, pl.ds(base, ROWS_PER_SUB)])

    return sc_kernel


def _make_sc_count():
    out_types = jax.ShapeDtypeStruct((NC, N_PAD, CW), jnp.float32)
    scratch = [pltpu.VMEM((CHUNK,), jnp.int32),
               pltpu.VMEM((CHUNK, CW), jnp.float32),
               pltpu.VMEM((CHUNK, CW), jnp.float32),
               pltpu.VMEM_SHARED((N_PAD, CW), jnp.float32)]

    @functools.partial(pl.kernel, out_type=out_types, mesh=_mesh,
                       scratch_types=scratch)
    def sc_kernel(dst_hbm, cc, idx_d, ones, zc, cnt):
        cid = lax.axis_index("c")
        sid = lax.axis_index("s")
        wid = sid * NC + cid
        base = sid * ROWS_PER_SUB

        @pl.loop(0, CHUNK)
        def _(r):
            for c in range(0, CW, LANES):
                ones[r, pl.ds(c, LANES)] = jnp.ones((LANES,), jnp.float32)
                zc[r, pl.ds(c, LANES)] = jnp.zeros((LANES,), jnp.float32)

        for k in range(ROWS_PER_SUB // CHUNK):
            pltpu.sync_copy(zc, cnt.at[pl.ds(base + k * CHUNK, CHUNK)])
        plsc.subcore_barrier()

        ebase = wid * CPT * CHUNK

        @pl.loop(0, CPT)
        def _(i):
            pltpu.sync_copy(dst_hbm.at[pl.ds(ebase + i * CHUNK, CHUNK)], idx_d)
            pltpu.sync_copy(ones, cnt.at[idx_d], add=True)

        plsc.subcore_barrier()

        for k in range(ROWS_PER_SUB // CHUNK):
            sl = pl.ds(base + k * CHUNK, CHUNK)
            pltpu.sync_copy(cnt.at[sl], cc.at[cid, sl])

    return sc_kernel


_sc_aggregate = _make_sc_aggregate()
_sc_count = _make_sc_count()

BM = 1000


def _tc_layer_body(p0, p1, c0, c1, h, wl, wr, b, o):
    invc = 1.0 / jnp.maximum(c0[:, 0:1] + c1[:, 0:1], 1.0)
    mean = (p0[...] + p1[...]) * invc
    hp = lax.dot_general(mean, wl[...], (((1,), (1,)), ((), ())),
                         precision=lax.Precision.HIGHEST)
    hp += lax.dot_general(h[...], wr[...], (((1,), (1,)), ((), ())),
                          precision=lax.Precision.HIGHEST)
    o[...] = jnp.maximum(hp + b[...], 0.0)


def _tc_layer2_body(p0, p1, c0, c1, h, wl, wr, b, wout, bout, o):
    invc = 1.0 / jnp.maximum(c0[:, 0:1] + c1[:, 0:1], 1.0)
    mean = (p0[...] + p1[...]) * invc
    hp = lax.dot_general(mean, wl[...], (((1,), (1,)), ((), ())),
                         precision=lax.Precision.HIGHEST)
    hp += lax.dot_general(h[...], wr[...], (((1,), (1,)), ((), ())),
                          precision=lax.Precision.HIGHEST)
    h2 = jnp.maximum(hp + b[...], 0.0)
    o[...] = lax.dot_general(h2, wout[...], (((1,), (1,)), ((), ())),
                             precision=lax.Precision.HIGHEST) + bout[...]


def _row_spec(bm, d):
    return pl.BlockSpec((bm, d), lambda i: (i, 0))


def _full_spec(shape):
    return pl.BlockSpec(shape, lambda i: tuple(0 for _ in shape))


def _tc_layer1(p0, p1, c0, c1, h, wl, wr, b):
    grid = (N // BM,)
    return pl.pallas_call(
        _tc_layer_body,
        grid=grid,
        in_specs=[_row_spec(BM, D), _row_spec(BM, D),
                  _row_spec(BM, CW), _row_spec(BM, CW),
                  _row_spec(BM, D),
                  _full_spec((D, D)), _full_spec((D, D)),
                  _full_spec((1, D))],
        out_specs=_row_spec(BM, D),
        out_shape=jax.ShapeDtypeStruct((N, D), jnp.float32),
    )(p0, p1, c0, c1, h, wl, wr, b.reshape(1, D))


def _tc_layer2(p0, p1, c0, c1, h, wl, wr, b, wout, bout):
    grid = (N // BM,)
    return pl.pallas_call(
        _tc_layer2_body,
        grid=grid,
        in_specs=[_row_spec(BM, D), _row_spec(BM, D),
                  _row_spec(BM, CW), _row_spec(BM, CW),
                  _row_spec(BM, D),
                  _full_spec((D, D)), _full_spec((D, D)),
                  _full_spec((1, D)),
                  _full_spec((P, D)), _full_spec((1, P))],
        out_specs=_row_spec(BM, P),
        out_shape=jax.ShapeDtypeStruct((N, P), jnp.float32),
    )(p0, p1, c0, c1, h, wl, wr, b.reshape(1, D), wout, bout.reshape(1, P))


def kernel(x, edge_index, Wl1, Wr1, b1, Wl2, Wr2, b2, Wout, bout):
    src = edge_index[0]
    dst = edge_index[1]
    pad = E_PAD - E
    ar = jnp.arange(pad, dtype=jnp.int32)
    src_p = jnp.concatenate([src, ar % N])
    dst_p = jnp.concatenate([dst, N + 8 + (ar % (N_PAD - N - 16))])

    zrows = jnp.zeros((ROWS_PER_SUB, D), jnp.float32)
    cc = _sc_count(dst_p)
    pp = _sc_aggregate(x, src_p, dst_p, zrows)
    h1 = _tc_layer1(pp[0], pp[1], cc[0], cc[1], x, Wl1, Wr1, b1)
    qq = _sc_aggregate(h1, src_p, dst_p, zrows)
    out = _tc_layer2(qq[0], qq[1], cc[0], cc[1], h1, Wl2, Wr2, b2, Wout, bout)
    return out


# speedup vs baseline: 9.2789x; 1.0563x over previous
import functools

import jax
import jax.numpy as jnp
from jax import lax
from jax.experimental import pallas as pl
from jax.experimental.pallas import tpu as pltpu
from jax.experimental.pallas import tpu_sc as plsc

N, E, D, P = 10000, 320000, 128, 12

NC, NS, LANES = 2, 16, 16
NW = NC * NS
CHUNK = 128
CPT = 80
E_PAD = NW * CPT * CHUNK
ROWS_PER_SUB = 640
N_PAD = NS * ROWS_PER_SUB
DUMMY_ROW = N_PAD - 8
CW = 128

_mesh = plsc.VectorSubcoreMesh(core_axis_name="c", subcore_axis_name="s")


def _make_sc_aggregate():
    out_types = jax.ShapeDtypeStruct((NC, N_PAD, D), jnp.float32)
    scratch = [pltpu.VMEM((CPT * CHUNK,), jnp.int32),
               pltpu.VMEM((CHUNK,), jnp.int32),
               pltpu.VMEM((CHUNK,), jnp.int32),
               pltpu.VMEM((CHUNK, D), jnp.float32),
               pltpu.VMEM((CHUNK, D), jnp.float32),
               pltpu.VMEM_SHARED((N_PAD, D), jnp.float32),
               pltpu.SemaphoreType.DMA,
               pltpu.SemaphoreType.DMA,
               pltpu.SemaphoreType.DMA,
               pltpu.SemaphoreType.DMA]

    @functools.partial(pl.kernel, out_type=out_types, mesh=_mesh,
                       scratch_types=scratch)
    def sc_kernel(h_hbm, src_hbm, dst_hbm, z_hbm, pp, srcall, dst0, dst1,
                  rows0, rows1, acc, sg0, sg1, sd0, sd1):
        cid = lax.axis_index("c")
        sid = lax.axis_index("s")
        wid = sid * NC + cid
        base = sid * ROWS_PER_SUB
        ebase = wid * CPT * CHUNK

        dstb = (dst0, dst1)
        rows = (rows0, rows1)
        sg = (sg0, sg1)
        sd = (sd0, sd1)

        pltpu.sync_copy(src_hbm.at[pl.ds(ebase, CPT * CHUNK)], srcall)
        pltpu.sync_copy(z_hbm, acc.at[pl.ds(base, ROWS_PER_SUB)])

        def issue(i, b):
            pltpu.async_copy(dst_hbm.at[pl.ds(ebase + i * CHUNK, CHUNK)],
                             dstb[b], sd[b])
            pltpu.async_copy(h_hbm.at[srcall.at[pl.ds(i * CHUNK, CHUNK)]],
                             rows[b], sg[b])

        issue(0, 0)
        issue(1, 1)
        plsc.subcore_barrier()

        @pl.loop(0, CPT // 2)
        def _(j):
            i0 = j * 2
            for b in range(2):
                i = i0 + b
                pltpu.make_async_copy(dst_hbm.at[pl.ds(0, CHUNK)],
                                      dstb[b], sd[b]).wait()
                pltpu.make_async_copy(h_hbm.at[pl.ds(0, CHUNK)],
                                      rows[b], sg[b]).wait()
                pltpu.sync_copy(rows[b], acc.at[dstb[b]], add=True)

                @pl.when(i + 2 < CPT)
                def _():
                    issue(i + 2, b)

        plsc.subcore_barrier()
        pltpu.sync_copy(acc.at[pl.ds(base, ROWS_PER_SUB)],
                        pp.at[cid, pl.ds(base, ROWS_PER_SUB)])

    return sc_kernel


def _make_sc_count():
    out_types = jax.ShapeDtypeStruct((NC, N_PAD, CW), jnp.float32)
    scratch = [pltpu.VMEM((CHUNK,), jnp.int32),
               pltpu.VMEM((CHUNK,), jnp.int32),
               pltpu.VMEM((CHUNK, CW), jnp.float32),
               pltpu.VMEM_SHARED((N_PAD, CW), jnp.float32),
               pltpu.SemaphoreType.DMA,
               pltpu.SemaphoreType.DMA]

    @functools.partial(pl.kernel, out_type=out_types, mesh=_mesh,
                       scratch_types=scratch)
    def sc_kernel(dst_hbm, ones_hbm, zc_hbm, cc, dst0, dst1, ones, cnt,
                  sd0, sd1):
        cid = lax.axis_index("c")
        sid = lax.axis_index("s")
        wid = sid * NC + cid
        base = sid * ROWS_PER_SUB
        ebase = wid * CPT * CHUNK

        dstb = (dst0, dst1)
        sd = (sd0, sd1)

        pltpu.sync_copy(ones_hbm, ones)
        pltpu.sync_copy(zc_hbm, cnt.at[pl.ds(base, ROWS_PER_SUB)])

        def issue(i, b):
            pltpu.async_copy(dst_hbm.at[pl.ds(ebase + i * CHUNK, CHUNK)],
                             dstb[b], sd[b])

        issue(0, 0)
        issue(1, 1)
        plsc.subcore_barrier()

        @pl.loop(0, CPT // 2)
        def _(j):
            i0 = j * 2
            for b in range(2):
                i = i0 + b
                pltpu.make_async_copy(dst_hbm.at[pl.ds(0, CHUNK)],
                                      dstb[b], sd[b]).wait()
                pltpu.sync_copy(ones, cnt.at[dstb[b]], add=True)

                @pl.when(i + 2 < CPT)
                def _():
                    issue(i + 2, b)

        plsc.subcore_barrier()
        pltpu.sync_copy(cnt.at[pl.ds(base, ROWS_PER_SUB)],
                        cc.at[cid, pl.ds(base, ROWS_PER_SUB)])

    return sc_kernel


_sc_aggregate = _make_sc_aggregate()
_sc_count = _make_sc_count()

BM = 1000


def _tc_layer_body(p0, p1, c0, c1, h, wl, wr, b, o):
    invc = 1.0 / jnp.maximum(c0[:, 0:1] + c1[:, 0:1], 1.0)
    mean = (p0[...] + p1[...]) * invc
    hp = lax.dot_general(mean, wl[...], (((1,), (1,)), ((), ())),
                         precision=lax.Precision.HIGHEST)
    hp += lax.dot_general(h[...], wr[...], (((1,), (1,)), ((), ())),
                          precision=lax.Precision.HIGHEST)
    o[...] = jnp.maximum(hp + b[...], 0.0)


def _tc_layer2_body(p0, p1, c0, c1, h, wl, wr, b, wout, bout, o):
    invc = 1.0 / jnp.maximum(c0[:, 0:1] + c1[:, 0:1], 1.0)
    mean = (p0[...] + p1[...]) * invc
    hp = lax.dot_general(mean, wl[...], (((1,), (1,)), ((), ())),
                         precision=lax.Precision.HIGHEST)
    hp += lax.dot_general(h[...], wr[...], (((1,), (1,)), ((), ())),
                          precision=lax.Precision.HIGHEST)
    h2 = jnp.maximum(hp + b[...], 0.0)
    o[...] = lax.dot_general(h2, wout[...], (((1,), (1,)), ((), ())),
                             precision=lax.Precision.HIGHEST) + bout[...]


def _row_spec(bm, d):
    return pl.BlockSpec((bm, d), lambda i: (i, 0))


def _full_spec(shape):
    return pl.BlockSpec(shape, lambda i: tuple(0 for _ in shape))


def _tc_layer1(p0, p1, c0, c1, h, wl, wr, b):
    grid = (N // BM,)
    return pl.pallas_call(
        _tc_layer_body,
        grid=grid,
        in_specs=[_row_spec(BM, D), _row_spec(BM, D),
                  _row_spec(BM, CW), _row_spec(BM, CW),
                  _row_spec(BM, D),
                  _full_spec((D, D)), _full_spec((D, D)),
                  _full_spec((1, D))],
        out_specs=_row_spec(BM, D),
        out_shape=jax.ShapeDtypeStruct((N, D), jnp.float32),
    )(p0, p1, c0, c1, h, wl, wr, b.reshape(1, D))


def _tc_layer2(p0, p1, c0, c1, h, wl, wr, b, wout, bout):
    grid = (N // BM,)
    return pl.pallas_call(
        _tc_layer2_body,
        grid=grid,
        in_specs=[_row_spec(BM, D), _row_spec(BM, D),
                  _row_spec(BM, CW), _row_spec(BM, CW),
                  _row_spec(BM, D),
                  _full_spec((D, D)), _full_spec((D, D)),
                  _full_spec((1, D)),
                  _full_spec((P, D)), _full_spec((1, P))],
        out_specs=_row_spec(BM, P),
        out_shape=jax.ShapeDtypeStruct((N, P), jnp.float32),
    )(p0, p1, c0, c1, h, wl, wr, b.reshape(1, D), wout, bout.reshape(1, P))


def kernel(x, edge_index, Wl1, Wr1, b1, Wl2, Wr2, b2, Wout, bout):
    src = edge_index[0]
    dst = edge_index[1]
    pad = E_PAD - E
    ar = jnp.arange(pad, dtype=jnp.int32)
    src_p = jnp.concatenate([src, ar % N])
    dst_p = jnp.concatenate([dst, N + 8 + (ar % (N_PAD - N - 16))])

    zrows = jnp.zeros((ROWS_PER_SUB, D), jnp.float32)
    ones_cw = jnp.ones((CHUNK, CW), jnp.float32)
    zc_cw = jnp.zeros((ROWS_PER_SUB, CW), jnp.float32)
    cc = _sc_count(dst_p, ones_cw, zc_cw)
    pp = _sc_aggregate(x, src_p, dst_p, zrows)
    h1 = _tc_layer1(pp[0], pp[1], cc[0], cc[1], x, Wl1, Wr1, b1)
    qq = _sc_aggregate(h1, src_p, dst_p, zrows)
    out = _tc_layer2(qq[0], qq[1], cc[0], cc[1], h1, Wl2, Wr2, b2, Wout, bout)
    return out


# speedup vs baseline: 10.1431x; 1.0931x over previous
import functools

import jax
import jax.numpy as jnp
from jax import lax
from jax.experimental import pallas as pl
from jax.experimental.pallas import tpu as pltpu
from jax.experimental.pallas import tpu_sc as plsc

N, E, D, P = 10000, 320000, 128, 12

NC, NS, LANES = 2, 16, 16
NW = NC * NS
CHUNK = 128
CPT = 80
E_PAD = NW * CPT * CHUNK
ROWS_PER_SUB = 640
N_PAD = NS * ROWS_PER_SUB
DUMMY_ROW = N_PAD - 8
CW = 128

_mesh = plsc.VectorSubcoreMesh(core_axis_name="c", subcore_axis_name="s")


def _make_sc_aggregate():
    out_types = jax.ShapeDtypeStruct((NC, N_PAD, D), jnp.float32)
    scratch = [pltpu.VMEM((CPT * CHUNK,), jnp.int32),
               pltpu.VMEM((CHUNK,), jnp.int32),
               pltpu.VMEM((CHUNK,), jnp.int32),
               pltpu.VMEM((CHUNK, D), jnp.float32),
               pltpu.VMEM((CHUNK, D), jnp.float32),
               pltpu.VMEM_SHARED((N_PAD, D), jnp.float32),
               pltpu.SemaphoreType.DMA,
               pltpu.SemaphoreType.DMA,
               pltpu.SemaphoreType.DMA,
               pltpu.SemaphoreType.DMA]

    @functools.partial(pl.kernel, out_type=out_types, mesh=_mesh,
                       scratch_types=scratch)
    def sc_kernel(h_hbm, src_hbm, dst_hbm, z_hbm, pp, srcall, dst0, dst1,
                  rows0, rows1, acc, sg0, sg1, sd0, sd1):
        cid = lax.axis_index("c")
        sid = lax.axis_index("s")
        wid = sid * NC + cid
        base = sid * ROWS_PER_SUB
        ebase = wid * CPT * CHUNK

        dstb = (dst0, dst1)
        rows = (rows0, rows1)
        sg = (sg0, sg1)
        sd = (sd0, sd1)

        pltpu.sync_copy(src_hbm.at[pl.ds(ebase, CPT * CHUNK)], srcall)
        pltpu.sync_copy(z_hbm, acc.at[pl.ds(base, ROWS_PER_SUB)])

        def issue(i, b):
            pltpu.async_copy(dst_hbm.at[pl.ds(ebase + i * CHUNK, CHUNK)],
                             dstb[b], sd[b])
            pltpu.async_copy(h_hbm.at[srcall.at[pl.ds(i * CHUNK, CHUNK)]],
                             rows[b], sg[b])

        issue(0, 0)
        issue(1, 1)
        plsc.subcore_barrier()

        @pl.loop(0, CPT // 2)
        def _(j):
            i0 = j * 2
            for b in range(2):
                i = i0 + b
                pltpu.make_async_copy(dst_hbm.at[pl.ds(0, CHUNK)],
                                      dstb[b], sd[b]).wait()
                pltpu.make_async_copy(h_hbm.at[pl.ds(0, CHUNK)],
                                      rows[b], sg[b]).wait()
                pltpu.sync_copy(rows[b], acc.at[dstb[b]], add=True)

                @pl.when(i + 2 < CPT)
                def _():
                    issue(i + 2, b)

        plsc.subcore_barrier()
        pltpu.sync_copy(acc.at[pl.ds(base, ROWS_PER_SUB)],
                        pp.at[cid, pl.ds(base, ROWS_PER_SUB)])

    return sc_kernel


def _make_sc_count():
    out_types = jax.ShapeDtypeStruct((NC, N_PAD, CW), jnp.float32)
    scratch = [pltpu.VMEM((CHUNK,), jnp.int32),
               pltpu.VMEM((CHUNK,), jnp.int32),
               pltpu.VMEM((CHUNK, CW), jnp.float32),
               pltpu.VMEM_SHARED((N_PAD, CW), jnp.float32),
               pltpu.SemaphoreType.DMA,
               pltpu.SemaphoreType.DMA]

    @functools.partial(pl.kernel, out_type=out_types, mesh=_mesh,
                       scratch_types=scratch)
    def sc_kernel(dst_hbm, ones_hbm, zc_hbm, cc, dst0, dst1, ones, cnt,
                  sd0, sd1):
        cid = lax.axis_index("c")
        sid = lax.axis_index("s")
        wid = sid * NC + cid
        base = sid * ROWS_PER_SUB
        ebase = wid * CPT * CHUNK

        dstb = (dst0, dst1)
        sd = (sd0, sd1)

        pltpu.sync_copy(ones_hbm, ones)
        pltpu.sync_copy(zc_hbm, cnt.at[pl.ds(base, ROWS_PER_SUB)])

        def issue(i, b):
            pltpu.async_copy(dst_hbm.at[pl.ds(ebase + i * CHUNK, CHUNK)],
                             dstb[b], sd[b])

        issue(0, 0)
        issue(1, 1)
        plsc.subcore_barrier()

        @pl.loop(0, CPT // 2)
        def _(j):
            i0 = j * 2
            for b in range(2):
                i = i0 + b
                pltpu.make_async_copy(dst_hbm.at[pl.ds(0, CHUNK)],
                                      dstb[b], sd[b]).wait()
                pltpu.sync_copy(ones, cnt.at[dstb[b]], add=True)

                @pl.when(i + 2 < CPT)
                def _():
                    issue(i + 2, b)

        plsc.subcore_barrier()
        pltpu.sync_copy(cnt.at[pl.ds(base, ROWS_PER_SUB)],
                        cc.at[cid, pl.ds(base, ROWS_PER_SUB)])

    return sc_kernel


_sc_aggregate = _make_sc_aggregate()
_sc_count = _make_sc_count()

BM = 1000


def _tc_layer_body(p0, p1, c0, c1, h, wl, wr, b, o):
    invc = 1.0 / jnp.maximum(c0[:, 0:1] + c1[:, 0:1], 1.0)
    mean = (p0[...] + p1[...]) * invc
    hp = lax.dot_general(mean, wl[...], (((1,), (1,)), ((), ())),
                         precision=lax.Precision.DEFAULT)
    hp += lax.dot_general(h[...], wr[...], (((1,), (1,)), ((), ())),
                          precision=lax.Precision.DEFAULT)
    o[...] = jnp.maximum(hp + b[...], 0.0)


def _tc_layer2_body(p0, p1, c0, c1, h, wl, wr, b, wout, bout, o):
    invc = 1.0 / jnp.maximum(c0[:, 0:1] + c1[:, 0:1], 1.0)
    mean = (p0[...] + p1[...]) * invc
    hp = lax.dot_general(mean, wl[...], (((1,), (1,)), ((), ())),
                         precision=lax.Precision.DEFAULT)
    hp += lax.dot_general(h[...], wr[...], (((1,), (1,)), ((), ())),
                          precision=lax.Precision.DEFAULT)
    h2 = jnp.maximum(hp + b[...], 0.0)
    o[...] = lax.dot_general(h2, wout[...], (((1,), (1,)), ((), ())),
                             precision=lax.Precision.DEFAULT) + bout[...]


def _row_spec(bm, d):
    return pl.BlockSpec((bm, d), lambda i: (i, 0))


def _full_spec(shape):
    return pl.BlockSpec(shape, lambda i: tuple(0 for _ in shape))


def _tc_layer1(p0, p1, c0, c1, h, wl, wr, b):
    grid = (N // BM,)
    return pl.pallas_call(
        _tc_layer_body,
        grid=grid,
        in_specs=[_row_spec(BM, D), _row_spec(BM, D),
                  _row_spec(BM, CW), _row_spec(BM, CW),
                  _row_spec(BM, D),
                  _full_spec((D, D)), _full_spec((D, D)),
                  _full_spec((1, D))],
        out_specs=_row_spec(BM, D),
        out_shape=jax.ShapeDtypeStruct((N, D), jnp.float32),
    )(p0, p1, c0, c1, h, wl, wr, b.reshape(1, D))


def _tc_layer2(p0, p1, c0, c1, h, wl, wr, b, wout, bout):
    grid = (N // BM,)
    return pl.pallas_call(
        _tc_layer2_body,
        grid=grid,
        in_specs=[_row_spec(BM, D), _row_spec(BM, D),
                  _row_spec(BM, CW), _row_spec(BM, CW),
                  _row_spec(BM, D),
                  _full_spec((D, D)), _full_spec((D, D)),
                  _full_spec((1, D)),
                  _full_spec((P, D)), _full_spec((1, P))],
        out_specs=_row_spec(BM, P),
        out_shape=jax.ShapeDtypeStruct((N, P), jnp.float32),
    )(p0, p1, c0, c1, h, wl, wr, b.reshape(1, D), wout, bout.reshape(1, P))


def kernel(x, edge_index, Wl1, Wr1, b1, Wl2, Wr2, b2, Wout, bout):
    src = edge_index[0]
    dst = edge_index[1]
    pad = E_PAD - E
    ar = jnp.arange(pad, dtype=jnp.int32)
    src_p = jnp.concatenate([src, ar % N])
    dst_p = jnp.concatenate([dst, N + 8 + (ar % (N_PAD - N - 16))])

    zrows = jnp.zeros((ROWS_PER_SUB, D), jnp.float32)
    ones_cw = jnp.ones((CHUNK, CW), jnp.float32)
    zc_cw = jnp.zeros((ROWS_PER_SUB, CW), jnp.float32)
    cc = _sc_count(dst_p, ones_cw, zc_cw)
    pp = _sc_aggregate(x, src_p, dst_p, zrows)
    h1 = _tc_layer1(pp[0], pp[1], cc[0], cc[1], x, Wl1, Wr1, b1)
    qq = _sc_aggregate(h1, src_p, dst_p, zrows)
    out = _tc_layer2(qq[0], qq[1], cc[0], cc[1], h1, Wl2, Wr2, b2, Wout, bout)
    return out
